# baseline (speedup 1.0000x reference)
import functools

import numpy as np
import jax
import jax.numpy as jnp
from jax import lax
from jax.experimental import pallas as pl
from jax.experimental.pallas import tpu as pltpu
from jax.experimental.pallas import tpu_sc as plsc

BASE_GSD = 10.0
EMBED = 768
N = EMBED // 4
T = 12
S = 3
B = 4
H = 16
W = 16
R = T * S
BH = B * H


def _pos_table() -> np.ndarray:
    omega = 1.0 / (10000.0 ** (np.arange(N // 2, dtype=np.float32) / (N / 2.0)))
    out = np.arange(T, dtype=np.float32)[:, None] * omega[None, :]
    tab = np.concatenate([np.sin(out), np.cos(out)], axis=-1).astype(np.float32)
    return np.repeat(tab, S, axis=0)


def _month_table() -> np.ndarray:
    angles = np.arange(13, dtype=np.float32) * (2.0 * np.pi / 12.0)
    sin_t = np.tile(np.sin(angles)[:, None], (1, N // 2))
    cos_t = np.tile(np.cos(angles)[:, None], (1, N // 2))
    return np.concatenate([sin_t[:-1], cos_t[:-1]], axis=-1).astype(np.float32)


_POS_REP = _pos_table()
_MONTH_TAB = _month_table()
_GATHER_W = 256
_MONTH_TAB_PAD = np.zeros((12, _GATHER_W), dtype=np.float32)
_MONTH_TAB_PAD[:, :N] = _MONTH_TAB
_OMEGA48 = (1.0 / (10000.0 ** (np.arange(48, dtype=np.float32) / 48.0))).astype(
    np.float32
)


_GATHER_ROWS = B * R
_ROWS_PER_WORKER = 8
_NUM_WORKERS = _GATHER_ROWS // _ROWS_PER_WORKER


def _sc_month_gather(table: jax.Array, idx: jax.Array) -> jax.Array:
    mesh = plsc.VectorSubcoreMesh(core_axis_name="c", subcore_axis_name="s")

    @functools.partial(
        pl.kernel,
        mesh=mesh,
        out_type=jax.ShapeDtypeStruct((_GATHER_ROWS, _GATHER_W), jnp.float32),
        scratch_types=[
            pltpu.VMEM((_ROWS_PER_WORKER,), jnp.int32),
            pltpu.VMEM((_ROWS_PER_WORKER, _GATHER_W), jnp.float32),
            pltpu.SemaphoreType.DMA,
        ],
    )
    def k(table_hbm, idx_hbm, out_hbm, idx_v, rows_v, sem):
        wid = lax.axis_index("s") * 2 + lax.axis_index("c")

        @pl.when(wid < _NUM_WORKERS)
        def _():
            base = wid * _ROWS_PER_WORKER
            pltpu.sync_copy(idx_hbm.at[pl.ds(base, _ROWS_PER_WORKER)], idx_v)
            pltpu.async_copy(table_hbm.at[idx_v], rows_v, sem).wait()
            pltpu.sync_copy(rows_v, out_hbm.at[pl.ds(base, _ROWS_PER_WORKER)])

    return k(table, idx)


_NBUF = 8
_NCH = BH


def _add_body(tok_hbm, add_v, sp_v, out_hbm, bin_ref, bout_ref, sem_in, sem_out):
    def start_in(i, slot):
        pltpu.make_async_copy(
            tok_hbm.at[i], bin_ref.at[slot], sem_in.at[slot]
        ).start()

    for k in range(_NBUF):
        start_in(k, k)

    def step(i, carry):
        slot = lax.rem(i, _NBUF)
        pltpu.make_async_copy(
            tok_hbm.at[i], bin_ref.at[slot], sem_in.at[slot]
        ).wait()
        prev = lax.max(i - _NBUF, 0)

        @pl.when(i >= _NBUF)
        def _():
            pltpu.make_async_copy(
                bout_ref.at[slot], out_hbm.at[prev], sem_out.at[slot]
            ).wait()

        a = add_v[i // H]
        sp = sp_v[lax.rem(i, H)]
        bout_ref[slot] = bin_ref[slot] + a[:, None, :] + sp[None, :, :]
        pltpu.make_async_copy(
            bout_ref.at[slot], out_hbm.at[i], sem_out.at[slot]
        ).start()

        nxt = i + _NBUF

        @pl.when(nxt < _NCH)
        def _():
            start_in(lax.min(nxt, _NCH - 1), slot)

        return carry

    lax.fori_loop(0, _NCH, step, 0)
    for k in range(_NBUF):
        i = _NCH - _NBUF + k
        pltpu.make_async_copy(
            bout_ref.at[i % _NBUF], out_hbm.at[i], sem_out.at[i % _NBUF]
        ).wait()


def _tc_add(tokv: jax.Array, addfull: jax.Array, sfull: jax.Array) -> jax.Array:
    return pl.pallas_call(
        _add_body,
        in_specs=[
            pl.BlockSpec(memory_space=pl.ANY),
            pl.BlockSpec(memory_space=pltpu.VMEM),
            pl.BlockSpec(memory_space=pltpu.VMEM),
        ],
        out_specs=pl.BlockSpec(memory_space=pl.ANY),
        out_shape=jax.ShapeDtypeStruct((BH, R, W, EMBED), jnp.float32),
        scratch_shapes=[
            pltpu.VMEM((_NBUF, R, W, EMBED), jnp.float32),
            pltpu.VMEM((_NBUF, R, W, EMBED), jnp.float32),
            pltpu.SemaphoreType.DMA((_NBUF,)),
            pltpu.SemaphoreType.DMA((_NBUF,)),
        ],
    )(tokv, addfull, sfull)


def kernel(tokens, timestamps, ch_embed, patch_size, input_res):
    b, h, w, t, s_, d = tokens.shape
    gsd = (
        jnp.asarray(input_res, jnp.float32) * jnp.asarray(patch_size, jnp.float32)
    ) / BASE_GSD

    months = timestamps[:, :, 1].astype(jnp.int32)
    m_idx = jnp.repeat(months, S, axis=1).reshape(-1)
    month_rep = _sc_month_gather(jnp.asarray(_MONTH_TAB_PAD), m_idx)
    month_rep = month_rep[:, :N].reshape(B, R, N)

    ch_rep = jnp.tile(ch_embed, (T, 1))
    addfull = jnp.concatenate(
        [
            jnp.broadcast_to(ch_rep[None], (B, R, N)),
            jnp.broadcast_to(jnp.asarray(_POS_REP)[None], (B, R, N)),
            month_rep,
            jnp.zeros((B, R, N), jnp.float32),
        ],
        axis=-1,
    )

    coords = jnp.arange(H, dtype=jnp.float32) * gsd
    ang = coords[:, None] * jnp.asarray(_OMEGA48)[None, :]
    e96 = jnp.concatenate([jnp.sin(ang), jnp.cos(ang)], axis=-1)
    sp = jnp.concatenate(
        [
            jnp.broadcast_to(e96[None, :, :], (H, W, 96)),
            jnp.broadcast_to(e96[:, None, :], (H, W, 96)),
        ],
        axis=-1,
    )
    sfull = jnp.concatenate(
        [jnp.zeros((H, W, EMBED - N), jnp.float32), sp], axis=-1
    )

    tokv = jnp.transpose(tokens, (0, 1, 3, 4, 2, 5)).reshape(BH, R, W, EMBED)
    out = _tc_add(tokv, addfull, sfull)
    out = out.reshape(B, H, T, S, W, EMBED).transpose(0, 1, 4, 2, 3, 5)
    return out


# speedup vs baseline: 1.0031x; 1.0031x over previous
import functools

import numpy as np
import jax
import jax.numpy as jnp
from jax import lax
from jax.experimental import pallas as pl
from jax.experimental.pallas import tpu as pltpu
from jax.experimental.pallas import tpu_sc as plsc

BASE_GSD = 10.0
EMBED = 768
N = EMBED // 4
T = 12
S = 3
B = 4
H = 16
W = 16
R = T * S
BH = B * H


def _pos_table() -> np.ndarray:
    omega = 1.0 / (10000.0 ** (np.arange(N // 2, dtype=np.float32) / (N / 2.0)))
    out = np.arange(T, dtype=np.float32)[:, None] * omega[None, :]
    tab = np.concatenate([np.sin(out), np.cos(out)], axis=-1).astype(np.float32)
    return np.repeat(tab, S, axis=0)


def _month_table() -> np.ndarray:
    angles = np.arange(13, dtype=np.float32) * (2.0 * np.pi / 12.0)
    sin_t = np.tile(np.sin(angles)[:, None], (1, N // 2))
    cos_t = np.tile(np.cos(angles)[:, None], (1, N // 2))
    return np.concatenate([sin_t[:-1], cos_t[:-1]], axis=-1).astype(np.float32)


_POS_REP = _pos_table()
_MONTH_TAB = _month_table()
_GATHER_W = 256
_MONTH_TAB_PAD = np.zeros((12, _GATHER_W), dtype=np.float32)
_MONTH_TAB_PAD[:, :N] = _MONTH_TAB
_OMEGA48 = (1.0 / (10000.0 ** (np.arange(48, dtype=np.float32) / 48.0))).astype(
    np.float32
)


_GATHER_ROWS = B * R
_ROWS_PER_WORKER = 8
_NUM_WORKERS = _GATHER_ROWS // _ROWS_PER_WORKER


def _sc_month_gather(table: jax.Array, idx: jax.Array) -> jax.Array:
    mesh = plsc.VectorSubcoreMesh(core_axis_name="c", subcore_axis_name="s")

    @functools.partial(
        pl.kernel,
        mesh=mesh,
        out_type=jax.ShapeDtypeStruct((_GATHER_ROWS, _GATHER_W), jnp.float32),
        scratch_types=[
            pltpu.VMEM((_ROWS_PER_WORKER,), jnp.int32),
            pltpu.VMEM((_ROWS_PER_WORKER, _GATHER_W), jnp.float32),
            pltpu.SemaphoreType.DMA,
        ],
    )
    def k(table_hbm, idx_hbm, out_hbm, idx_v, rows_v, sem):
        wid = lax.axis_index("s") * 2 + lax.axis_index("c")

        @pl.when(wid < _NUM_WORKERS)
        def _():
            base = wid * _ROWS_PER_WORKER
            pltpu.sync_copy(idx_hbm.at[pl.ds(base, _ROWS_PER_WORKER)], idx_v)
            pltpu.async_copy(table_hbm.at[idx_v], rows_v, sem).wait()
            pltpu.sync_copy(rows_v, out_hbm.at[pl.ds(base, _ROWS_PER_WORKER)])

    return k(table, idx)


_NBUF = 4
_NCH = BH // 2


def _add_body(tok_hbm, add_v, sp_v, out_hbm, bin_ref, bout_ref, sem_in, sem_out):
    def start_in(i, slot):
        pltpu.make_async_copy(
            tok_hbm.at[i], bin_ref.at[slot], sem_in.at[slot]
        ).start()

    for k in range(_NBUF):
        start_in(k, k)

    def step(i, carry):
        slot = lax.rem(i, _NBUF)
        pltpu.make_async_copy(
            tok_hbm.at[i], bin_ref.at[slot], sem_in.at[slot]
        ).wait()
        prev = lax.max(i - _NBUF, 0)

        @pl.when(i >= _NBUF)
        def _():
            pltpu.make_async_copy(
                bout_ref.at[slot], out_hbm.at[prev], sem_out.at[slot]
            ).wait()

        a = add_v[i // (H // 2)]
        sp0 = sp_v[2 * lax.rem(i, H // 2)]
        sp1 = sp_v[2 * lax.rem(i, H // 2) + 1]
        x = bin_ref[slot]
        bout_ref[slot, : R] = x[:R] + a[:, None, :] + sp0[None, :, :]
        bout_ref[slot, R:] = x[R:] + a[:, None, :] + sp1[None, :, :]
        pltpu.make_async_copy(
            bout_ref.at[slot], out_hbm.at[i], sem_out.at[slot]
        ).start()

        nxt = i + _NBUF

        @pl.when(nxt < _NCH)
        def _():
            start_in(lax.min(nxt, _NCH - 1), slot)

        return carry

    lax.fori_loop(0, _NCH, step, 0)
    for k in range(_NBUF):
        i = _NCH - _NBUF + k
        pltpu.make_async_copy(
            bout_ref.at[i % _NBUF], out_hbm.at[i], sem_out.at[i % _NBUF]
        ).wait()


def _tc_add(tokv: jax.Array, addfull: jax.Array, sfull: jax.Array) -> jax.Array:
    return pl.pallas_call(
        _add_body,
        in_specs=[
            pl.BlockSpec(memory_space=pl.ANY),
            pl.BlockSpec(memory_space=pltpu.VMEM),
            pl.BlockSpec(memory_space=pltpu.VMEM),
        ],
        out_specs=pl.BlockSpec(memory_space=pl.ANY),
        out_shape=jax.ShapeDtypeStruct((_NCH, 2 * R, W, EMBED), jnp.float32),
        scratch_shapes=[
            pltpu.VMEM((_NBUF, 2 * R, W, EMBED), jnp.float32),
            pltpu.VMEM((_NBUF, 2 * R, W, EMBED), jnp.float32),
            pltpu.SemaphoreType.DMA((_NBUF,)),
            pltpu.SemaphoreType.DMA((_NBUF,)),
        ],
    )(tokv, addfull, sfull)


def kernel(tokens, timestamps, ch_embed, patch_size, input_res):
    b, h, w, t, s_, d = tokens.shape
    gsd = (
        jnp.asarray(input_res, jnp.float32) * jnp.asarray(patch_size, jnp.float32)
    ) / BASE_GSD

    months = timestamps[:, :, 1].astype(jnp.int32)
    m_idx = jnp.repeat(months, S, axis=1).reshape(-1)
    month_rep = _sc_month_gather(jnp.asarray(_MONTH_TAB_PAD), m_idx)
    month_rep = month_rep[:, :N].reshape(B, R, N)

    ch_rep = jnp.tile(ch_embed, (T, 1))
    addfull = jnp.concatenate(
        [
            jnp.broadcast_to(ch_rep[None], (B, R, N)),
            jnp.broadcast_to(jnp.asarray(_POS_REP)[None], (B, R, N)),
            month_rep,
            jnp.zeros((B, R, N), jnp.float32),
        ],
        axis=-1,
    )

    coords = jnp.arange(H, dtype=jnp.float32) * gsd
    ang = coords[:, None] * jnp.asarray(_OMEGA48)[None, :]
    e96 = jnp.concatenate([jnp.sin(ang), jnp.cos(ang)], axis=-1)
    sp = jnp.concatenate(
        [
            jnp.broadcast_to(e96[None, :, :], (H, W, 96)),
            jnp.broadcast_to(e96[:, None, :], (H, W, 96)),
        ],
        axis=-1,
    )
    sfull = jnp.concatenate(
        [jnp.zeros((H, W, EMBED - N), jnp.float32), sp], axis=-1
    )

    tokv = jnp.transpose(tokens, (0, 1, 3, 4, 2, 5)).reshape(BH // 2, 2 * R, W, EMBED)
    out = _tc_add(tokv, addfull, sfull)
    out = out.reshape(B, H, T, S, W, EMBED).transpose(0, 1, 4, 2, 3, 5)
    return out


# speedup vs baseline: 1.0447x; 1.0415x over previous
import functools

import numpy as np
import jax
import jax.numpy as jnp
from jax import lax
from jax.experimental import pallas as pl
from jax.experimental.pallas import tpu as pltpu
from jax.experimental.pallas import tpu_sc as plsc

BASE_GSD = 10.0
EMBED = 768
N = EMBED // 4
T = 12
S = 3
B = 4
H = 16
W = 16
R = T * S
BH = B * H


def _pos_table() -> np.ndarray:
    omega = 1.0 / (10000.0 ** (np.arange(N // 2, dtype=np.float32) / (N / 2.0)))
    out = np.arange(T, dtype=np.float32)[:, None] * omega[None, :]
    tab = np.concatenate([np.sin(out), np.cos(out)], axis=-1).astype(np.float32)
    return np.repeat(tab, S, axis=0)


def _month_table() -> np.ndarray:
    angles = np.arange(13, dtype=np.float32) * (2.0 * np.pi / 12.0)
    sin_t = np.tile(np.sin(angles)[:, None], (1, N // 2))
    cos_t = np.tile(np.cos(angles)[:, None], (1, N // 2))
    return np.concatenate([sin_t[:-1], cos_t[:-1]], axis=-1).astype(np.float32)


_POS_REP = _pos_table()
_MONTH_TAB = _month_table()
_GATHER_W = 256
_MONTH_TAB_PAD = np.zeros((12, _GATHER_W), dtype=np.float32)
_MONTH_TAB_PAD[:, :N] = _MONTH_TAB
_OMEGA48 = (1.0 / (10000.0 ** (np.arange(48, dtype=np.float32) / 48.0))).astype(
    np.float32
)


_GATHER_ROWS = B * R
_ROWS_PER_WORKER = 8
_NUM_WORKERS = _GATHER_ROWS // _ROWS_PER_WORKER


def _sc_month_gather(table: jax.Array, idx: jax.Array) -> jax.Array:
    mesh = plsc.VectorSubcoreMesh(core_axis_name="c", subcore_axis_name="s")

    @functools.partial(
        pl.kernel,
        mesh=mesh,
        out_type=jax.ShapeDtypeStruct((_GATHER_ROWS, _GATHER_W), jnp.float32),
        scratch_types=[
            pltpu.VMEM((_ROWS_PER_WORKER,), jnp.int32),
            pltpu.VMEM((_ROWS_PER_WORKER, _GATHER_W), jnp.float32),
            pltpu.SemaphoreType.DMA,
        ],
    )
    def k(table_hbm, idx_hbm, out_hbm, idx_v, rows_v, sem):
        wid = lax.axis_index("s") * 2 + lax.axis_index("c")

        @pl.when(wid < _NUM_WORKERS)
        def _():
            base = wid * _ROWS_PER_WORKER
            pltpu.sync_copy(idx_hbm.at[pl.ds(base, _ROWS_PER_WORKER)], idx_v)
            pltpu.async_copy(table_hbm.at[idx_v], rows_v, sem).wait()
            pltpu.sync_copy(rows_v, out_hbm.at[pl.ds(base, _ROWS_PER_WORKER)])

    return k(table, idx)


_NBUF = 4
_NCH = BH // 2


def _add_body(
    tok_hbm, month_v, ch_v, pos_v, om_v, gsd_s, out_hbm,
    add_v, sp_v, bin_ref, bout_ref, sem_in, sem_out,
):
    def start_in(i, slot):
        pltpu.make_async_copy(
            tok_hbm.at[i], bin_ref.at[slot], sem_in.at[slot]
        ).start()

    for k in range(_NBUF):
        start_in(k, k)

    ch = ch_v[...]
    ch_rep = jnp.concatenate([ch] * T, axis=0)
    pos_c = pos_v[...]
    z36 = jnp.zeros((R, N), jnp.float32)
    for bi in range(B):
        m = month_v[pl.ds(bi * R, R), pl.ds(0, N)]
        add_v[bi] = jnp.concatenate([ch_rep, pos_c, m, z36], axis=1)

    g = gsd_s[0]
    om = om_v[...]
    hw_idx = lax.broadcasted_iota(jnp.int32, (H, 48), 0).astype(jnp.float32)
    ang = hw_idx * g * om
    e96 = jnp.concatenate([jnp.sin(ang), jnp.cos(ang)], axis=1)
    z576 = jnp.zeros((W, EMBED - N), jnp.float32)
    for hi in range(H):
        eh = jnp.broadcast_to(e96[hi : hi + 1, :], (W, 96))
        sp_v[hi] = jnp.concatenate([z576, e96, eh], axis=1)

    def step(i, carry):
        slot = lax.rem(i, _NBUF)
        pltpu.make_async_copy(
            tok_hbm.at[i], bin_ref.at[slot], sem_in.at[slot]
        ).wait()
        prev = lax.max(i - _NBUF, 0)

        @pl.when(i >= _NBUF)
        def _():
            pltpu.make_async_copy(
                bout_ref.at[slot], out_hbm.at[prev], sem_out.at[slot]
            ).wait()

        a = add_v[i // (H // 2)]
        sp0 = sp_v[2 * lax.rem(i, H // 2)]
        sp1 = sp_v[2 * lax.rem(i, H // 2) + 1]
        x = bin_ref[slot]
        bout_ref[slot, : R] = x[:R] + a[:, None, :] + sp0[None, :, :]
        bout_ref[slot, R:] = x[R:] + a[:, None, :] + sp1[None, :, :]
        pltpu.make_async_copy(
            bout_ref.at[slot], out_hbm.at[i], sem_out.at[slot]
        ).start()

        nxt = i + _NBUF

        @pl.when(nxt < _NCH)
        def _():
            start_in(lax.min(nxt, _NCH - 1), slot)

        return carry

    lax.fori_loop(0, _NCH, step, 0)
    for k in range(_NBUF):
        i = _NCH - _NBUF + k
        pltpu.make_async_copy(
            bout_ref.at[i % _NBUF], out_hbm.at[i], sem_out.at[i % _NBUF]
        ).wait()


def _tc_add(tokv, month_rep, ch_embed, gsd) -> jax.Array:
    return pl.pallas_call(
        _add_body,
        in_specs=[
            pl.BlockSpec(memory_space=pl.ANY),
            pl.BlockSpec(memory_space=pltpu.VMEM),
            pl.BlockSpec(memory_space=pltpu.VMEM),
            pl.BlockSpec(memory_space=pltpu.VMEM),
            pl.BlockSpec(memory_space=pltpu.VMEM),
            pl.BlockSpec(memory_space=pltpu.SMEM),
        ],
        out_specs=pl.BlockSpec(memory_space=pl.ANY),
        out_shape=jax.ShapeDtypeStruct((_NCH, 2 * R, W, EMBED), jnp.float32),
        scratch_shapes=[
            pltpu.VMEM((B, R, EMBED), jnp.float32),
            pltpu.VMEM((H, W, EMBED), jnp.float32),
            pltpu.VMEM((_NBUF, 2 * R, W, EMBED), jnp.float32),
            pltpu.VMEM((_NBUF, 2 * R, W, EMBED), jnp.float32),
            pltpu.SemaphoreType.DMA((_NBUF,)),
            pltpu.SemaphoreType.DMA((_NBUF,)),
        ],
    )(
        tokv,
        month_rep,
        ch_embed,
        jnp.asarray(_POS_REP),
        jnp.asarray(_OMEGA48).reshape(1, 48),
        gsd,
    )


def kernel(tokens, timestamps, ch_embed, patch_size, input_res):
    b, h, w, t, s_, d = tokens.shape
    gsd = (
        jnp.asarray(input_res, jnp.float32) * jnp.asarray(patch_size, jnp.float32)
    ) / BASE_GSD

    months = timestamps[:, :, 1].astype(jnp.int32)
    m_idx = jnp.repeat(months, S, axis=1).reshape(-1)
    month_rep = _sc_month_gather(jnp.asarray(_MONTH_TAB_PAD), m_idx)

    tokv = jnp.transpose(tokens, (0, 1, 3, 4, 2, 5)).reshape(BH // 2, 2 * R, W, EMBED)
    out = _tc_add(tokv, month_rep, ch_embed, gsd.reshape(1))
    out = out.reshape(B, H, T, S, W, EMBED).transpose(0, 1, 4, 2, 3, 5)
    return out


# speedup vs baseline: 1.0524x; 1.0074x over previous
import functools

import numpy as np
import jax
import jax.numpy as jnp
from jax import lax
from jax.experimental import pallas as pl
from jax.experimental.pallas import tpu as pltpu
from jax.experimental.pallas import tpu_sc as plsc

BASE_GSD = 10.0
EMBED = 768
N = EMBED // 4
T = 12
S = 3
B = 4
H = 16
W = 16
R = T * S
BH = B * H


def _pos_table() -> np.ndarray:
    omega = 1.0 / (10000.0 ** (np.arange(N // 2, dtype=np.float32) / (N / 2.0)))
    out = np.arange(T, dtype=np.float32)[:, None] * omega[None, :]
    tab = np.concatenate([np.sin(out), np.cos(out)], axis=-1).astype(np.float32)
    return np.repeat(tab, S, axis=0)


def _month_table() -> np.ndarray:
    angles = np.arange(13, dtype=np.float32) * (2.0 * np.pi / 12.0)
    sin_t = np.tile(np.sin(angles)[:, None], (1, N // 2))
    cos_t = np.tile(np.cos(angles)[:, None], (1, N // 2))
    return np.concatenate([sin_t[:-1], cos_t[:-1]], axis=-1).astype(np.float32)


_POS_REP = _pos_table()
_MONTH_TAB = _month_table()
_GATHER_W = 256
_MONTH_TAB_PAD = np.zeros((12, _GATHER_W), dtype=np.float32)
_MONTH_TAB_PAD[:, :N] = _MONTH_TAB
_OMEGA48 = (1.0 / (10000.0 ** (np.arange(48, dtype=np.float32) / 48.0))).astype(
    np.float32
)


_GATHER_ROWS = B * T
_ROWS_PER_WORKER = 8
_NUM_WORKERS = _GATHER_ROWS // _ROWS_PER_WORKER


def _sc_month_gather(table: jax.Array, idx: jax.Array) -> jax.Array:
    mesh = plsc.VectorSubcoreMesh(core_axis_name="c", subcore_axis_name="s")

    @functools.partial(
        pl.kernel,
        mesh=mesh,
        out_type=jax.ShapeDtypeStruct((_GATHER_ROWS, _GATHER_W), jnp.float32),
        scratch_types=[
            pltpu.VMEM((_ROWS_PER_WORKER,), jnp.int32),
            pltpu.VMEM((_ROWS_PER_WORKER, _GATHER_W), jnp.float32),
            pltpu.SemaphoreType.DMA,
        ],
    )
    def k(table_hbm, idx_hbm, out_hbm, idx_v, rows_v, sem):
        wid = lax.axis_index("s") * 2 + lax.axis_index("c")

        @pl.when(wid < _NUM_WORKERS)
        def _():
            base = wid * _ROWS_PER_WORKER
            pltpu.sync_copy(idx_hbm.at[pl.ds(base, _ROWS_PER_WORKER)], idx_v)
            pltpu.async_copy(table_hbm.at[idx_v], rows_v, sem).wait()
            pltpu.sync_copy(rows_v, out_hbm.at[pl.ds(base, _ROWS_PER_WORKER)])

    return k(table, idx)


_NBUF = 4
_NCH = BH // 2


def _add_body(
    tok_hbm, month_v, ch_v, pos_v, om_v, gsd_s, out_hbm,
    add_v, sp_v, bin_ref, bout_ref, sem_in, sem_out,
):
    def start_in(i, slot):
        pltpu.make_async_copy(
            tok_hbm.at[i], bin_ref.at[slot], sem_in.at[slot]
        ).start()

    for k in range(_NBUF):
        start_in(k, k)

    ch = ch_v[...]
    ch_rep = jnp.concatenate([ch] * T, axis=0)
    pos_c = pos_v[...]
    z36 = jnp.zeros((R, N), jnp.float32)
    for bi in range(B):
        m12 = month_v[pl.ds(bi * T, T), pl.ds(0, N)]
        m = jnp.concatenate(
            [m12[ti : ti + 1] for ti in range(T) for _ in range(S)], axis=0
        )
        add_v[bi] = jnp.concatenate([ch_rep, pos_c, m, z36], axis=1)

    g = gsd_s[0]
    om = om_v[...]
    hw_idx = lax.broadcasted_iota(jnp.int32, (H, 48), 0).astype(jnp.float32)
    ang = hw_idx * g * om
    e96 = jnp.concatenate([jnp.sin(ang), jnp.cos(ang)], axis=1)
    z576 = jnp.zeros((W, EMBED - N), jnp.float32)
    for hi in range(H):
        eh = jnp.broadcast_to(e96[hi : hi + 1, :], (W, 96))
        sp_v[hi] = jnp.concatenate([z576, e96, eh], axis=1)

    def step(i, carry):
        slot = lax.rem(i, _NBUF)
        pltpu.make_async_copy(
            tok_hbm.at[i], bin_ref.at[slot], sem_in.at[slot]
        ).wait()
        prev = lax.max(i - _NBUF, 0)

        @pl.when(i >= _NBUF)
        def _():
            pltpu.make_async_copy(
                bout_ref.at[slot], out_hbm.at[prev], sem_out.at[slot]
            ).wait()

        a = add_v[i // (H // 2)]
        sp0 = sp_v[2 * lax.rem(i, H // 2)]
        sp1 = sp_v[2 * lax.rem(i, H // 2) + 1]
        x = bin_ref[slot]
        bout_ref[slot, : R] = x[:R] + a[:, None, :] + sp0[None, :, :]
        bout_ref[slot, R:] = x[R:] + a[:, None, :] + sp1[None, :, :]
        pltpu.make_async_copy(
            bout_ref.at[slot], out_hbm.at[i], sem_out.at[slot]
        ).start()

        nxt = i + _NBUF

        @pl.when(nxt < _NCH)
        def _():
            start_in(lax.min(nxt, _NCH - 1), slot)

        return carry

    lax.fori_loop(0, _NCH, step, 0)
    for k in range(_NBUF):
        i = _NCH - _NBUF + k
        pltpu.make_async_copy(
            bout_ref.at[i % _NBUF], out_hbm.at[i], sem_out.at[i % _NBUF]
        ).wait()


def _tc_add(tokv, month_rep, ch_embed, gsd) -> jax.Array:
    return pl.pallas_call(
        _add_body,
        in_specs=[
            pl.BlockSpec(memory_space=pl.ANY),
            pl.BlockSpec(memory_space=pltpu.VMEM),
            pl.BlockSpec(memory_space=pltpu.VMEM),
            pl.BlockSpec(memory_space=pltpu.VMEM),
            pl.BlockSpec(memory_space=pltpu.VMEM),
            pl.BlockSpec(memory_space=pltpu.SMEM),
        ],
        out_specs=pl.BlockSpec(memory_space=pl.ANY),
        out_shape=jax.ShapeDtypeStruct((_NCH, 2 * R, W, EMBED), jnp.float32),
        scratch_shapes=[
            pltpu.VMEM((B, R, EMBED), jnp.float32),
            pltpu.VMEM((H, W, EMBED), jnp.float32),
            pltpu.VMEM((_NBUF, 2 * R, W, EMBED), jnp.float32),
            pltpu.VMEM((_NBUF, 2 * R, W, EMBED), jnp.float32),
            pltpu.SemaphoreType.DMA((_NBUF,)),
            pltpu.SemaphoreType.DMA((_NBUF,)),
        ],
    )(
        tokv,
        month_rep,
        ch_embed,
        jnp.asarray(_POS_REP),
        jnp.asarray(_OMEGA48).reshape(1, 48),
        gsd,
    )


def kernel(tokens, timestamps, ch_embed, patch_size, input_res):
    b, h, w, t, s_, d = tokens.shape
    gsd = (
        jnp.asarray(input_res, jnp.float32) * jnp.asarray(patch_size, jnp.float32)
    ) / BASE_GSD

    m_idx = timestamps[:, :, 1].astype(jnp.int32).reshape(-1)
    month_rep = _sc_month_gather(jnp.asarray(_MONTH_TAB_PAD), m_idx)

    tokv = jnp.transpose(tokens, (0, 1, 3, 4, 2, 5)).reshape(BH // 2, 2 * R, W, EMBED)
    out = _tc_add(tokv, month_rep, ch_embed, gsd.reshape(1))
    out = out.reshape(B, H, T, S, W, EMBED).transpose(0, 1, 4, 2, 3, 5)
    return out


# speedup vs baseline: 1.0671x; 1.0139x over previous
import functools

import numpy as np
import jax
import jax.numpy as jnp
from jax import lax
from jax.experimental import pallas as pl
from jax.experimental.pallas import tpu as pltpu
from jax.experimental.pallas import tpu_sc as plsc

BASE_GSD = 10.0
EMBED = 768
N = EMBED // 4
T = 12
S = 3
B = 4
H = 16
W = 16
R = T * S
BH = B * H


def _pos_table() -> np.ndarray:
    omega = 1.0 / (10000.0 ** (np.arange(N // 2, dtype=np.float32) / (N / 2.0)))
    out = np.arange(T, dtype=np.float32)[:, None] * omega[None, :]
    tab = np.concatenate([np.sin(out), np.cos(out)], axis=-1).astype(np.float32)
    return np.repeat(tab, S, axis=0)


def _month_table() -> np.ndarray:
    angles = np.arange(13, dtype=np.float32) * (2.0 * np.pi / 12.0)
    sin_t = np.tile(np.sin(angles)[:, None], (1, N // 2))
    cos_t = np.tile(np.cos(angles)[:, None], (1, N // 2))
    return np.concatenate([sin_t[:-1], cos_t[:-1]], axis=-1).astype(np.float32)


_POS_REP = _pos_table()
_MONTH_TAB = _month_table()
_GATHER_W = 256
_MONTH_TAB_PAD = np.zeros((12, _GATHER_W), dtype=np.float32)
_MONTH_TAB_PAD[:, :N] = _MONTH_TAB
_OMEGA48 = (1.0 / (10000.0 ** (np.arange(48, dtype=np.float32) / 48.0))).astype(
    np.float32
)


_GATHER_ROWS = B * T
_ROWS_PER_WORKER = 8
_NUM_WORKERS = _GATHER_ROWS // _ROWS_PER_WORKER


def _sc_month_gather(table: jax.Array, idx: jax.Array) -> jax.Array:
    _MESH_CORES = 1
    mesh = plsc.VectorSubcoreMesh(
        core_axis_name="c", subcore_axis_name="s", num_cores=_MESH_CORES
    )

    @functools.partial(
        pl.kernel,
        mesh=mesh,
        out_type=jax.ShapeDtypeStruct((_GATHER_ROWS, _GATHER_W), jnp.float32),
        scratch_types=[
            pltpu.VMEM((_ROWS_PER_WORKER,), jnp.int32),
            pltpu.VMEM((_ROWS_PER_WORKER, _GATHER_W), jnp.float32),
            pltpu.SemaphoreType.DMA,
        ],
    )
    def k(table_hbm, idx_hbm, out_hbm, idx_v, rows_v, sem):
        wid = lax.axis_index("s") * _MESH_CORES + lax.axis_index("c")

        @pl.when(wid < _NUM_WORKERS)
        def _():
            base = wid * _ROWS_PER_WORKER
            pltpu.sync_copy(idx_hbm.at[pl.ds(base, _ROWS_PER_WORKER)], idx_v)
            pltpu.async_copy(table_hbm.at[idx_v], rows_v, sem).wait()
            pltpu.sync_copy(rows_v, out_hbm.at[pl.ds(base, _ROWS_PER_WORKER)])

    return k(table, idx)


_NBUF = 4
_NCH = BH // 2


def _add_body(
    tok_hbm, month_v, ch_v, pos_v, om_v, gsd_s, out_hbm,
    add_v, sp_v, bin_ref, bout_ref, sem_in, sem_out,
):
    def start_in(i, slot):
        pltpu.make_async_copy(
            tok_hbm.at[i], bin_ref.at[slot], sem_in.at[slot]
        ).start()

    for k in range(_NBUF):
        start_in(k, k)

    ch = ch_v[...]
    ch_rep = jnp.concatenate([ch] * T, axis=0)
    pos_c = pos_v[...]
    z36 = jnp.zeros((R, N), jnp.float32)
    for bi in range(B):
        m12 = month_v[pl.ds(bi * T, T), pl.ds(0, N)]
        m = jnp.concatenate(
            [m12[ti : ti + 1] for ti in range(T) for _ in range(S)], axis=0
        )
        add_v[bi] = jnp.concatenate([ch_rep, pos_c, m, z36], axis=1)

    g = gsd_s[0]
    om = om_v[...]
    hw_idx = lax.broadcasted_iota(jnp.int32, (H, 48), 0).astype(jnp.float32)
    ang = hw_idx * g * om
    e96 = jnp.concatenate([jnp.sin(ang), jnp.cos(ang)], axis=1)
    z576 = jnp.zeros((W, EMBED - N), jnp.float32)
    for hi in range(H):
        eh = jnp.broadcast_to(e96[hi : hi + 1, :], (W, 96))
        sp_v[hi] = jnp.concatenate([z576, e96, eh], axis=1)

    def step(i, carry):
        slot = lax.rem(i, _NBUF)
        pltpu.make_async_copy(
            tok_hbm.at[i], bin_ref.at[slot], sem_in.at[slot]
        ).wait()
        prev = lax.max(i - _NBUF, 0)

        @pl.when(i >= _NBUF)
        def _():
            pltpu.make_async_copy(
                bout_ref.at[slot], out_hbm.at[prev], sem_out.at[slot]
            ).wait()

        a = add_v[i // (H // 2)]
        sp0 = sp_v[2 * lax.rem(i, H // 2)]
        sp1 = sp_v[2 * lax.rem(i, H // 2) + 1]
        x = bin_ref[slot]
        bout_ref[slot, : R] = x[:R] + a[:, None, :] + sp0[None, :, :]
        bout_ref[slot, R:] = x[R:] + a[:, None, :] + sp1[None, :, :]
        pltpu.make_async_copy(
            bout_ref.at[slot], out_hbm.at[i], sem_out.at[slot]
        ).start()

        nxt = i + _NBUF

        @pl.when(nxt < _NCH)
        def _():
            start_in(lax.min(nxt, _NCH - 1), slot)

        return carry

    lax.fori_loop(0, _NCH, step, 0)
    for k in range(_NBUF):
        i = _NCH - _NBUF + k
        pltpu.make_async_copy(
            bout_ref.at[i % _NBUF], out_hbm.at[i], sem_out.at[i % _NBUF]
        ).wait()


def _tc_add(tokv, month_rep, ch_embed, gsd) -> jax.Array:
    return pl.pallas_call(
        _add_body,
        in_specs=[
            pl.BlockSpec(memory_space=pl.ANY),
            pl.BlockSpec(memory_space=pltpu.VMEM),
            pl.BlockSpec(memory_space=pltpu.VMEM),
            pl.BlockSpec(memory_space=pltpu.VMEM),
            pl.BlockSpec(memory_space=pltpu.VMEM),
            pl.BlockSpec(memory_space=pltpu.SMEM),
        ],
        out_specs=pl.BlockSpec(memory_space=pl.ANY),
        out_shape=jax.ShapeDtypeStruct((_NCH, 2 * R, W, EMBED), jnp.float32),
        scratch_shapes=[
            pltpu.VMEM((B, R, EMBED), jnp.float32),
            pltpu.VMEM((H, W, EMBED), jnp.float32),
            pltpu.VMEM((_NBUF, 2 * R, W, EMBED), jnp.float32),
            pltpu.VMEM((_NBUF, 2 * R, W, EMBED), jnp.float32),
            pltpu.SemaphoreType.DMA((_NBUF,)),
            pltpu.SemaphoreType.DMA((_NBUF,)),
        ],
    )(
        tokv,
        month_rep,
        ch_embed,
        jnp.asarray(_POS_REP),
        jnp.asarray(_OMEGA48).reshape(1, 48),
        gsd,
    )


def kernel(tokens, timestamps, ch_embed, patch_size, input_res):
    b, h, w, t, s_, d = tokens.shape
    gsd = (
        jnp.asarray(input_res, jnp.float32) * jnp.asarray(patch_size, jnp.float32)
    ) / BASE_GSD

    m_idx = timestamps[:, :, 1].astype(jnp.int32).reshape(-1)
    month_rep = _sc_month_gather(jnp.asarray(_MONTH_TAB_PAD), m_idx)

    tokv = jnp.transpose(tokens, (0, 1, 3, 4, 2, 5)).reshape(BH // 2, 2 * R, W, EMBED)
    out = _tc_add(tokv, month_rep, ch_embed, gsd.reshape(1))
    out = out.reshape(B, H, T, S, W, EMBED).transpose(0, 1, 4, 2, 3, 5)
    return out


# speedup vs baseline: 1.0693x; 1.0021x over previous
import functools

import numpy as np
import jax
import jax.numpy as jnp
from jax import lax
from jax.experimental import pallas as pl
from jax.experimental.pallas import tpu as pltpu
from jax.experimental.pallas import tpu_sc as plsc

BASE_GSD = 10.0
EMBED = 768
N = EMBED // 4
T = 12
S = 3
B = 4
H = 16
W = 16
R = T * S
BH = B * H


def _pos_table() -> np.ndarray:
    omega = 1.0 / (10000.0 ** (np.arange(N // 2, dtype=np.float32) / (N / 2.0)))
    out = np.arange(T, dtype=np.float32)[:, None] * omega[None, :]
    tab = np.concatenate([np.sin(out), np.cos(out)], axis=-1).astype(np.float32)
    return np.repeat(tab, S, axis=0)


def _month_table() -> np.ndarray:
    angles = np.arange(13, dtype=np.float32) * (2.0 * np.pi / 12.0)
    sin_t = np.tile(np.sin(angles)[:, None], (1, N // 2))
    cos_t = np.tile(np.cos(angles)[:, None], (1, N // 2))
    return np.concatenate([sin_t[:-1], cos_t[:-1]], axis=-1).astype(np.float32)


_POS_REP = _pos_table()
_MONTH_TAB = _month_table()
_GATHER_W = 256
_MONTH_TAB_PAD = np.zeros((12, _GATHER_W), dtype=np.float32)
_MONTH_TAB_PAD[:, :N] = _MONTH_TAB
_OMEGA48 = (1.0 / (10000.0 ** (np.arange(48, dtype=np.float32) / 48.0))).astype(
    np.float32
)


_GATHER_ROWS = B * T
_ROWS_PER_WORKER = 8
_NUM_WORKERS = _GATHER_ROWS // _ROWS_PER_WORKER


def _sc_month_gather(table: jax.Array, idx: jax.Array) -> jax.Array:
    _MESH_CORES = 1
    mesh = plsc.VectorSubcoreMesh(
        core_axis_name="c", subcore_axis_name="s", num_cores=_MESH_CORES
    )

    @functools.partial(
        pl.kernel,
        mesh=mesh,
        out_type=jax.ShapeDtypeStruct((_GATHER_ROWS, _GATHER_W), jnp.float32),
        scratch_types=[
            pltpu.VMEM((_ROWS_PER_WORKER,), jnp.int32),
            pltpu.VMEM((_ROWS_PER_WORKER, _GATHER_W), jnp.float32),
            pltpu.SemaphoreType.DMA,
        ],
    )
    def k(table_hbm, idx_hbm, out_hbm, idx_v, rows_v, sem):
        wid = lax.axis_index("s") * _MESH_CORES + lax.axis_index("c")

        @pl.when(wid < _NUM_WORKERS)
        def _():
            base = wid * _ROWS_PER_WORKER
            pltpu.sync_copy(idx_hbm.at[pl.ds(base, _ROWS_PER_WORKER)], idx_v)
            pltpu.async_copy(table_hbm.at[idx_v], rows_v, sem).wait()
            pltpu.sync_copy(rows_v, out_hbm.at[pl.ds(base, _ROWS_PER_WORKER)])

    return k(table, idx)


_NBUF = 6
_NCH = BH // 2


def _add_body(
    tok_hbm, month_v, ch_v, pos_v, om_v, gsd_s, out_hbm,
    add_v, sp_v, bin_ref, bout_ref, sem_in, sem_out,
):
    def start_in(i, slot):
        pltpu.make_async_copy(
            tok_hbm.at[i], bin_ref.at[slot], sem_in.at[slot]
        ).start()

    for k in range(_NBUF):
        start_in(k, k)

    ch = ch_v[...]
    ch_rep = jnp.concatenate([ch] * T, axis=0)
    pos_c = pos_v[...]
    z36 = jnp.zeros((R, N), jnp.float32)
    for bi in range(B):
        m12 = month_v[pl.ds(bi * T, T), pl.ds(0, N)]
        m = jnp.concatenate(
            [m12[ti : ti + 1] for ti in range(T) for _ in range(S)], axis=0
        )
        add_v[bi] = jnp.concatenate([ch_rep, pos_c, m, z36], axis=1)

    g = gsd_s[0]
    om = om_v[...]
    hw_idx = lax.broadcasted_iota(jnp.int32, (H, 48), 0).astype(jnp.float32)
    ang = hw_idx * g * om
    e96 = jnp.concatenate([jnp.sin(ang), jnp.cos(ang)], axis=1)
    z576 = jnp.zeros((W, EMBED - N), jnp.float32)
    for hi in range(H):
        eh = jnp.broadcast_to(e96[hi : hi + 1, :], (W, 96))
        sp_v[hi] = jnp.concatenate([z576, e96, eh], axis=1)

    def step(i, carry):
        slot = lax.rem(i, _NBUF)
        pltpu.make_async_copy(
            tok_hbm.at[i], bin_ref.at[slot], sem_in.at[slot]
        ).wait()
        prev = lax.max(i - _NBUF, 0)

        @pl.when(i >= _NBUF)
        def _():
            pltpu.make_async_copy(
                bout_ref.at[slot], out_hbm.at[prev], sem_out.at[slot]
            ).wait()

        a = add_v[i // (H // 2)]
        sp0 = sp_v[2 * lax.rem(i, H // 2)]
        sp1 = sp_v[2 * lax.rem(i, H // 2) + 1]
        x = bin_ref[slot]
        bout_ref[slot, : R] = x[:R] + a[:, None, :] + sp0[None, :, :]
        bout_ref[slot, R:] = x[R:] + a[:, None, :] + sp1[None, :, :]
        pltpu.make_async_copy(
            bout_ref.at[slot], out_hbm.at[i], sem_out.at[slot]
        ).start()

        nxt = i + _NBUF

        @pl.when(nxt < _NCH)
        def _():
            start_in(lax.min(nxt, _NCH - 1), slot)

        return carry

    lax.fori_loop(0, _NCH, step, 0)
    for k in range(_NBUF):
        i = _NCH - _NBUF + k
        pltpu.make_async_copy(
            bout_ref.at[i % _NBUF], out_hbm.at[i], sem_out.at[i % _NBUF]
        ).wait()


def _tc_add(tokv, month_rep, ch_embed, gsd) -> jax.Array:
    return pl.pallas_call(
        _add_body,
        in_specs=[
            pl.BlockSpec(memory_space=pl.ANY),
            pl.BlockSpec(memory_space=pltpu.VMEM),
            pl.BlockSpec(memory_space=pltpu.VMEM),
            pl.BlockSpec(memory_space=pltpu.VMEM),
            pl.BlockSpec(memory_space=pltpu.VMEM),
            pl.BlockSpec(memory_space=pltpu.SMEM),
        ],
        out_specs=pl.BlockSpec(memory_space=pl.ANY),
        out_shape=jax.ShapeDtypeStruct((_NCH, 2 * R, W, EMBED), jnp.float32),
        scratch_shapes=[
            pltpu.VMEM((B, R, EMBED), jnp.float32),
            pltpu.VMEM((H, W, EMBED), jnp.float32),
            pltpu.VMEM((_NBUF, 2 * R, W, EMBED), jnp.float32),
            pltpu.VMEM((_NBUF, 2 * R, W, EMBED), jnp.float32),
            pltpu.SemaphoreType.DMA((_NBUF,)),
            pltpu.SemaphoreType.DMA((_NBUF,)),
        ],
    )(
        tokv,
        month_rep,
        ch_embed,
        jnp.asarray(_POS_REP),
        jnp.asarray(_OMEGA48).reshape(1, 48),
        gsd,
    )


def kernel(tokens, timestamps, ch_embed, patch_size, input_res):
    b, h, w, t, s_, d = tokens.shape
    gsd = (
        jnp.asarray(input_res, jnp.float32) * jnp.asarray(patch_size, jnp.float32)
    ) / BASE_GSD

    m_idx = timestamps[:, :, 1].astype(jnp.int32).reshape(-1)
    month_rep = _sc_month_gather(jnp.asarray(_MONTH_TAB_PAD), m_idx)

    tokv = jnp.transpose(tokens, (0, 1, 3, 4, 2, 5)).reshape(BH // 2, 2 * R, W, EMBED)
    out = _tc_add(tokv, month_rep, ch_embed, gsd.reshape(1))
    out = out.reshape(B, H, T, S, W, EMBED).transpose(0, 1, 4, 2, 3, 5)
    return out


# speedup vs baseline: 1.0714x; 1.0020x over previous
import functools

import numpy as np
import jax
import jax.numpy as jnp
from jax import lax
from jax.experimental import pallas as pl
from jax.experimental.pallas import tpu as pltpu
from jax.experimental.pallas import tpu_sc as plsc

BASE_GSD = 10.0
EMBED = 768
N = EMBED // 4
T = 12
S = 3
B = 4
H = 16
W = 16
R = T * S
BH = B * H


def _pos_table() -> np.ndarray:
    omega = 1.0 / (10000.0 ** (np.arange(N // 2, dtype=np.float32) / (N / 2.0)))
    out = np.arange(T, dtype=np.float32)[:, None] * omega[None, :]
    tab = np.concatenate([np.sin(out), np.cos(out)], axis=-1).astype(np.float32)
    return np.repeat(tab, S, axis=0)


def _month_table() -> np.ndarray:
    angles = np.arange(13, dtype=np.float32) * (2.0 * np.pi / 12.0)
    sin_t = np.tile(np.sin(angles)[:, None], (1, N // 2))
    cos_t = np.tile(np.cos(angles)[:, None], (1, N // 2))
    return np.concatenate([sin_t[:-1], cos_t[:-1]], axis=-1).astype(np.float32)


_POS_REP = _pos_table()
_MONTH_TAB = _month_table()
_GATHER_W = 256
_MONTH_TAB_PAD = np.zeros((12, _GATHER_W), dtype=np.float32)
_MONTH_TAB_PAD[:, :N] = _MONTH_TAB
_OMEGA48 = (1.0 / (10000.0 ** (np.arange(48, dtype=np.float32) / 48.0))).astype(
    np.float32
)


_GATHER_ROWS = B * T
_ROWS_PER_WORKER = 8
_NUM_WORKERS = _GATHER_ROWS // _ROWS_PER_WORKER


def _sc_month_gather(table: jax.Array, idx: jax.Array) -> jax.Array:
    _MESH_CORES = 1
    mesh = plsc.VectorSubcoreMesh(
        core_axis_name="c", subcore_axis_name="s", num_cores=_MESH_CORES
    )

    @functools.partial(
        pl.kernel,
        mesh=mesh,
        out_type=jax.ShapeDtypeStruct((_GATHER_ROWS, _GATHER_W), jnp.float32),
        scratch_types=[
            pltpu.VMEM((_ROWS_PER_WORKER,), jnp.int32),
            pltpu.VMEM((_ROWS_PER_WORKER, _GATHER_W), jnp.float32),
            pltpu.SemaphoreType.DMA,
        ],
    )
    def k(table_hbm, idx_hbm, out_hbm, idx_v, rows_v, sem):
        wid = lax.axis_index("s") * _MESH_CORES + lax.axis_index("c")

        @pl.when(wid < _NUM_WORKERS)
        def _():
            base = wid * _ROWS_PER_WORKER
            pltpu.sync_copy(idx_hbm.at[pl.ds(base, _ROWS_PER_WORKER)], idx_v)
            pltpu.async_copy(table_hbm.at[idx_v], rows_v, sem).wait()
            pltpu.sync_copy(rows_v, out_hbm.at[pl.ds(base, _ROWS_PER_WORKER)])

    return k(table, idx)


_NBUF = 3
_NCH = BH // 4


def _add_body(
    tok_hbm, month_v, ch_v, pos_v, om_v, gsd_s, out_hbm,
    add_v, sp_v, bin_ref, bout_ref, sem_in, sem_out,
):
    def start_in(i, slot):
        pltpu.make_async_copy(
            tok_hbm.at[i], bin_ref.at[slot], sem_in.at[slot]
        ).start()

    for k in range(_NBUF):
        start_in(k, k)

    ch = ch_v[...]
    ch_rep = jnp.concatenate([ch] * T, axis=0)
    pos_c = pos_v[...]
    z36 = jnp.zeros((R, N), jnp.float32)
    for bi in range(B):
        m12 = month_v[pl.ds(bi * T, T), pl.ds(0, N)]
        m = jnp.concatenate(
            [m12[ti : ti + 1] for ti in range(T) for _ in range(S)], axis=0
        )
        add_v[bi] = jnp.concatenate([ch_rep, pos_c, m, z36], axis=1)

    g = gsd_s[0]
    om = om_v[...]
    hw_idx = lax.broadcasted_iota(jnp.int32, (H, 48), 0).astype(jnp.float32)
    ang = hw_idx * g * om
    e96 = jnp.concatenate([jnp.sin(ang), jnp.cos(ang)], axis=1)
    z576 = jnp.zeros((W, EMBED - N), jnp.float32)
    for hi in range(H):
        eh = jnp.broadcast_to(e96[hi : hi + 1, :], (W, 96))
        sp_v[hi] = jnp.concatenate([z576, e96, eh], axis=1)

    def step(i, carry):
        slot = lax.rem(i, _NBUF)
        pltpu.make_async_copy(
            tok_hbm.at[i], bin_ref.at[slot], sem_in.at[slot]
        ).wait()
        prev = lax.max(i - _NBUF, 0)

        @pl.when(i >= _NBUF)
        def _():
            pltpu.make_async_copy(
                bout_ref.at[slot], out_hbm.at[prev], sem_out.at[slot]
            ).wait()

        a = add_v[i // (H // 4)]
        x = bin_ref[slot]
        for q in range(4):
            spq = sp_v[4 * lax.rem(i, H // 4) + q]
            bout_ref[slot, q * R : (q + 1) * R] = (
                x[q * R : (q + 1) * R] + a[:, None, :] + spq[None, :, :]
            )
        pltpu.make_async_copy(
            bout_ref.at[slot], out_hbm.at[i], sem_out.at[slot]
        ).start()

        nxt = i + _NBUF

        @pl.when(nxt < _NCH)
        def _():
            start_in(lax.min(nxt, _NCH - 1), slot)

        return carry

    lax.fori_loop(0, _NCH, step, 0)
    for k in range(_NBUF):
        i = _NCH - _NBUF + k
        pltpu.make_async_copy(
            bout_ref.at[i % _NBUF], out_hbm.at[i], sem_out.at[i % _NBUF]
        ).wait()


def _tc_add(tokv, month_rep, ch_embed, gsd) -> jax.Array:
    return pl.pallas_call(
        _add_body,
        in_specs=[
            pl.BlockSpec(memory_space=pl.ANY),
            pl.BlockSpec(memory_space=pltpu.VMEM),
            pl.BlockSpec(memory_space=pltpu.VMEM),
            pl.BlockSpec(memory_space=pltpu.VMEM),
            pl.BlockSpec(memory_space=pltpu.VMEM),
            pl.BlockSpec(memory_space=pltpu.SMEM),
        ],
        out_specs=pl.BlockSpec(memory_space=pl.ANY),
        out_shape=jax.ShapeDtypeStruct((_NCH, 4 * R, W, EMBED), jnp.float32),
        scratch_shapes=[
            pltpu.VMEM((B, R, EMBED), jnp.float32),
            pltpu.VMEM((H, W, EMBED), jnp.float32),
            pltpu.VMEM((_NBUF, 4 * R, W, EMBED), jnp.float32),
            pltpu.VMEM((_NBUF, 4 * R, W, EMBED), jnp.float32),
            pltpu.SemaphoreType.DMA((_NBUF,)),
            pltpu.SemaphoreType.DMA((_NBUF,)),
        ],
    )(
        tokv,
        month_rep,
        ch_embed,
        jnp.asarray(_POS_REP),
        jnp.asarray(_OMEGA48).reshape(1, 48),
        gsd,
    )


def kernel(tokens, timestamps, ch_embed, patch_size, input_res):
    b, h, w, t, s_, d = tokens.shape
    gsd = (
        jnp.asarray(input_res, jnp.float32) * jnp.asarray(patch_size, jnp.float32)
    ) / BASE_GSD

    m_idx = timestamps[:, :, 1].astype(jnp.int32).reshape(-1)
    month_rep = _sc_month_gather(jnp.asarray(_MONTH_TAB_PAD), m_idx)

    tokv = jnp.transpose(tokens, (0, 1, 3, 4, 2, 5)).reshape(BH // 4, 4 * R, W, EMBED)
    out = _tc_add(tokv, month_rep, ch_embed, gsd.reshape(1))
    out = out.reshape(B, H, T, S, W, EMBED).transpose(0, 1, 4, 2, 3, 5)
    return out
